# speedup vs baseline: 1.8405x; 1.8405x over previous
import functools
import operator

import numpy as np
import jax
import jax.numpy as jnp
from jax.experimental import pallas as pl

_C, _T, _H, _W = 3, 32, 384, 384
_ALPHA = 4
_NSLOW = _T // _ALPHA
_IDX = tuple(int(v) for v in np.linspace(0.0, _T - 1, _NSLOW).astype(np.float32))


def _body(in_ref, slow_ref, fast_ref):
    t = pl.program_id(0)
    x = in_ref[...]
    fast_ref[...] = x
    sel = functools.reduce(operator.or_, [t == i for i in _IDX])

    @pl.when(sel)
    def _():
        slow_ref[...] = x


def _slow_index_map(t):
    slot = sum((t >= i).astype(jnp.int32) for i in _IDX[1:])
    return (0, slot, 0, 0)


def kernel(frames):
    slow, fast = pl.pallas_call(
        _body,
        grid=(_T,),
        in_specs=[pl.BlockSpec((_C, 1, _H, _W), lambda t: (0, t, 0, 0))],
        out_specs=[
            pl.BlockSpec((_C, 1, _H, _W), _slow_index_map),
            pl.BlockSpec((_C, 1, _H, _W), lambda t: (0, t, 0, 0)),
        ],
        out_shape=[
            jax.ShapeDtypeStruct((_C, _NSLOW, _H, _W), frames.dtype),
            jax.ShapeDtypeStruct((_C, _T, _H, _W), frames.dtype),
        ],
    )(frames)
    return (slow, fast)
